# baseline (speedup 1.0000x reference)
import functools

import jax
import jax.numpy as jnp
from jax import lax
from jax.experimental import pallas as pl
from jax.experimental.pallas import tpu as pltpu
from jax.experimental.pallas import tpu_sc as plsc

N = 10000
E = 320000
G = 64
D = 32

NC = 2
NS = 16
NW = NC * NS
CH = 128
PERW = 80
NCHT = NW * PERW
EPAD = NCHT * CH - E
RPT = 632
KB = 16
HB = KB // 2
NP = RPT * NS


def _segsum_body(h_hbm, edges_hbm, out_hbm,
                 acc_sh, h_sh, src_v, dst_v, rows_a, zblk, gsem, psem, sem):
    c = lax.axis_index("c")
    s = lax.axis_index("s")
    wid = c * NS + s
    cbase = wid * PERW

    hd = pltpu.async_copy(h_hbm.at[pl.ds(s * RPT, RPT)],
                          h_sh.at[pl.ds(s * RPT, RPT)], psem.at[0])
    ed = pltpu.async_copy(edges_hbm.at[pl.ds(cbase, PERW)], src_v, psem.at[1])

    def zrow(i, carry):
        zblk[i, pl.ds(0, 16)] = jnp.zeros((16,), jnp.float32)
        zblk[i, pl.ds(16, 16)] = jnp.zeros((16,), jnp.float32)
        return carry

    lax.fori_loop(0, CH, zrow, 0, unroll=False)
    for r in range(4):
        pltpu.sync_copy(zblk, acc_sh.at[pl.ds(s * RPT + r * CH, CH)])
    pltpu.sync_copy(zblk.at[pl.ds(0, RPT - 4 * CH)],
                    acc_sh.at[pl.ds(s * RPT + 4 * CH, RPT - 4 * CH)])
    ed.wait()

    def unpack(i, carry):
        for k in range(CH // 16):
            p = src_v[i, pl.ds(16 * k, 16)]
            src_v[i, pl.ds(16 * k, 16)] = lax.bitwise_and(p, 0xFFFF)
            dst_v[i, pl.ds(16 * k, 16)] = lax.shift_right_logical(p, 16)
        return carry

    lax.fori_loop(0, PERW, unpack, 0, unroll=False)
    hd.wait()

    plsc.subcore_barrier()

    def fire_half(g, off):
        for b in range(HB):
            pltpu.async_copy(h_sh.at[src_v.at[g * HB + b]],
                             rows_a.at[off + b], gsem.at[off + b])

    def drain_half(g, off):
        sds = []
        for b in range(HB):
            pltpu.make_async_copy(h_sh.at[src_v.at[g * HB + b]],
                                  rows_a.at[off + b],
                                  gsem.at[off + b]).wait()
            sds.append(
                pltpu.async_copy(rows_a.at[off + b],
                                 acc_sh.at[dst_v.at[g * HB + b]],
                                 sem, add=True))
        return sds

    NG2 = PERW // KB
    fire_half(0, 0)

    def ring(j, carry):
        fire_half(2 * j + 1, HB)
        for sd in drain_half(2 * j, 0):
            sd.wait()

        @pl.when(j < NG2 - 1)
        def _():
            fire_half(2 * j + 2, 0)

        for sd in drain_half(2 * j + 1, HB):
            sd.wait()
        return carry

    lax.fori_loop(0, NG2, ring, 0, unroll=False)

    plsc.subcore_barrier()

    pltpu.sync_copy(acc_sh.at[pl.ds(s * RPT, RPT)],
                    out_hbm.at[c, pl.ds(s * RPT, RPT)])


@functools.partial(jax.jit, static_argnums=())
def _segsum(h, edges):
    mesh = plsc.VectorSubcoreMesh(
        core_axis_name="c", subcore_axis_name="s",
        num_cores=NC, num_subcores=NS)
    fn = pl.kernel(
        _segsum_body,
        out_type=jax.ShapeDtypeStruct((NC, NP, D), jnp.float32),
        mesh=mesh,
        scratch_types=[
            pltpu.VMEM_SHARED((NP, D), jnp.float32),
            pltpu.VMEM_SHARED((NP, D), jnp.float32),
            pltpu.VMEM((PERW, CH), jnp.int32),
            pltpu.VMEM((PERW, CH), jnp.int32),
            pltpu.VMEM((KB, CH, D), jnp.float32),
            pltpu.VMEM((CH, D), jnp.float32),
            pltpu.SemaphoreType.DMA((KB,)),
            pltpu.SemaphoreType.DMA((2,)),
            pltpu.SemaphoreType.DMA,
        ],
        compiler_params=pltpu.CompilerParams(use_tc_tiling_on_sc=False),
    )
    return fn(h, edges)


PK = 4
PR = N // PK
PRP = NP // PK


def _fold_norm_relu(u, dh, g_t, be_t):
    L = u.shape[1]
    us = u[:PR]
    csum = jnp.sum(us, axis=0, keepdims=True)
    ii = lax.broadcasted_iota(jnp.int32, (L, L), 0) % dh
    jj = lax.broadcasted_iota(jnp.int32, (L, L), 1) % dh
    fold = (ii == jj).astype(jnp.float32)
    mu = jnp.dot(csum, fold, preferred_element_type=jnp.float32) / N
    d = u - mu
    ds = d[:PR]
    c2 = jnp.sum(ds * ds, axis=0, keepdims=True)
    var = jnp.dot(c2, fold, preferred_element_type=jnp.float32) / N
    return jnp.maximum(d / jnp.sqrt(var + 1e-5) * g_t + be_t, 0.0)


def _proj_body(x_ref, w_ref, o_ref):
    o_ref[...] = jnp.dot(x_ref[...], w_ref[...],
                         preferred_element_type=jnp.float32)


def _proj(x_pad, w_bd):
    return pl.pallas_call(
        _proj_body,
        out_shape=jax.ShapeDtypeStruct((PRP, PK * D), jnp.float32),
    )(x_pad, w_bd)


def _mlp1_body(y_ref, agg_ref, eps_ref, ba_ref, g_ref, be_ref, wb_ref,
               bb_ref, o_ref):
    u = ((1.0 + eps_ref[0, 0]) * y_ref[...] + agg_ref[0] + agg_ref[1]
         + ba_ref[...])
    h = _fold_norm_relu(u, D, g_ref[...], be_ref[...])
    o_ref[...] = jnp.dot(h, wb_ref[...],
                         preferred_element_type=jnp.float32) + bb_ref[...]


def _mlp1(y, agg, eps, ba_t, g_t, be_t, wb_bd, bb_t):
    return pl.pallas_call(
        _mlp1_body,
        out_shape=jax.ShapeDtypeStruct((PRP, PK * D), jnp.float32),
    )(y, agg, eps.reshape(1, 1), ba_t, g_t, be_t, wb_bd, bb_t)


def _mlp_body(h_ref, agg_ref, eps_ref, wa_ref, ba_ref, g_ref, be_ref,
              wb_ref, bb_ref, o_ref):
    t = (1.0 + eps_ref[0, 0]) * h_ref[...] + agg_ref[0] + agg_ref[1]
    y = jnp.dot(t, wa_ref[...],
                preferred_element_type=jnp.float32) + ba_ref[...]
    h = _fold_norm_relu(y, 64, g_ref[...], be_ref[...])
    o_ref[...] = jnp.dot(h, wb_ref[...],
                         preferred_element_type=jnp.float32) + bb_ref[...]


def _mlp(h, agg, eps, wa_bd, ba_t, g_t, be_t, wb_bd, bb_t):
    return pl.pallas_call(
        _mlp_body,
        out_shape=jax.ShapeDtypeStruct((PRP, PK * D), jnp.float32),
    )(h, agg, eps.reshape(1, 1), wa_bd, ba_t, g_t, be_t, wb_bd, bb_t)


def _final_body(h_ref, agg_ref, eps_ref, wa_ref, ba_ref, g_ref, be_ref,
                wb_ref, bb_ref, batch_ref, wl_ref, bl_ref, o_ref):
    t = (1.0 + eps_ref[0, 0]) * h_ref[...] + agg_ref[0] + agg_ref[1]
    y = jnp.dot(t, wa_ref[...],
                preferred_element_type=jnp.float32) + ba_ref[...]
    h = _fold_norm_relu(y, 64, g_ref[...], be_ref[...])
    h4 = jnp.dot(h, wb_ref[...],
                 preferred_element_type=jnp.float32) + bb_ref[...]
    gids = lax.broadcasted_iota(jnp.int32, (PR, G), 1)
    sums = jnp.zeros((G, 16), jnp.float32)
    counts = jnp.zeros((G, 1), jnp.float32)
    for k in range(PK):
        oh = (batch_ref[:, k:k + 1] == gids).astype(jnp.float32)
        sums = sums + lax.dot_general(
            oh, h4[:PR, 16 * k:16 * k + 16], (((0,), (0,)), ((), ())),
            preferred_element_type=jnp.float32)
        counts = counts + jnp.sum(oh, axis=0)[:, None]
    pooled = sums / jnp.maximum(counts, 1.0)
    logit = jnp.dot(pooled, wl_ref[...],
                    preferred_element_type=jnp.float32) + bl_ref[...]
    o_ref[...] = jax.nn.sigmoid(logit)


def _final(h, agg, eps, wa_bd, ba_t, g_t, be_t, wb_bd, bb_t, batch_p,
           Wl, bl):
    return pl.pallas_call(
        _final_body,
        out_shape=jax.ShapeDtypeStruct((G, 1), jnp.float32),
    )(h, agg, eps.reshape(1, 1), wa_bd, ba_t, g_t, be_t, wb_bd, bb_t,
      batch_p, Wl, bl.reshape(1, 1))


def _bd(W):
    return jnp.kron(jnp.eye(PK, dtype=jnp.float32), W)


def _t4(v):
    return jnp.tile(v, PK)[None, :]


def kernel(x, edge_index, batch, eps1, Wa1, ba1, g1, be1, Wb1, bb1,
           eps2, Wa2, ba2, g2, be2, Wb2, bb2,
           eps3, Wa3, ba3, g3, be3, Wb3, bb3,
           eps4, Wa4, ba4, g4, be4, Wb4, bb4, Wl, bl):
    packed = jnp.bitwise_or(edge_index[0],
                            jnp.left_shift(edge_index[1], 16))
    pad = jnp.full((EPAD,), N << 16, jnp.int32)
    edges = jnp.concatenate([packed, pad]).reshape(NCHT, CH)
    x_pad = jnp.concatenate(
        [x, jnp.zeros((NP - N, x.shape[1]), jnp.float32)]).reshape(PRP, -1)
    batch_p = batch.reshape(PR, PK)

    y1 = _proj(x_pad, _bd(Wa1))
    a1 = _segsum(y1.reshape(NP, D), edges)
    h1 = _mlp1(y1, a1.reshape(NC, PRP, PK * D), eps1, _t4(ba1), _t4(g1),
               _t4(be1), _bd(Wb1), _t4(bb1))

    a2 = _segsum(h1.reshape(NP, D), edges)
    h2 = _mlp(h1, a2.reshape(NC, PRP, PK * D), eps2, _bd(Wa2), _t4(ba2),
              _t4(g2), _t4(be2), _bd(Wb2), _t4(bb2))

    a3 = _segsum(h2.reshape(NP, D), edges)
    h3 = _mlp(h2, a3.reshape(NC, PRP, PK * D), eps3, _bd(Wa3), _t4(ba3),
              _t4(g3), _t4(be3), _bd(Wb3), _t4(bb3))

    a4 = _segsum(h3.reshape(NP, D), edges)
    return _final(h3, a4.reshape(NC, PRP, PK * D), eps4, _bd(Wa4),
                  _t4(ba4), _t4(g4), _t4(be4), _bd(Wb4), _t4(bb4),
                  batch_p, Wl, bl)


# speedup vs baseline: 1.0273x; 1.0273x over previous
import functools

import jax
import jax.numpy as jnp
from jax import lax
from jax.experimental import pallas as pl
from jax.experimental.pallas import tpu as pltpu
from jax.experimental.pallas import tpu_sc as plsc

N = 10000
E = 320000
G = 64
D = 32

NC = 2
NS = 16
NW = NC * NS
CH = 128
PERW = 80
NCHT = NW * PERW
EPAD = NCHT * CH - E
RPT = 632
KB = 16
HB = KB // 2
NP = RPT * NS


def _segsum_body(h_hbm, edges_hbm, out_hbm,
                 acc_sh, h_sh, src_v, dst_v, rows_a, zblk, gsem, psem, sem):
    c = lax.axis_index("c")
    s = lax.axis_index("s")
    wid = c * NS + s
    cbase = wid * PERW

    hd = pltpu.async_copy(h_hbm.at[pl.ds(s * RPT, RPT)],
                          h_sh.at[pl.ds(s * RPT, RPT)], psem.at[0])
    ed = pltpu.async_copy(edges_hbm.at[pl.ds(cbase, PERW)], src_v, psem.at[1])

    def zrow(i, carry):
        zblk[i, pl.ds(0, 16)] = jnp.zeros((16,), jnp.float32)
        zblk[i, pl.ds(16, 16)] = jnp.zeros((16,), jnp.float32)
        return carry

    lax.fori_loop(0, CH, zrow, 0, unroll=False)
    for r in range(4):
        pltpu.sync_copy(zblk, acc_sh.at[pl.ds(s * RPT + r * CH, CH)])
    pltpu.sync_copy(zblk.at[pl.ds(0, RPT - 4 * CH)],
                    acc_sh.at[pl.ds(s * RPT + 4 * CH, RPT - 4 * CH)])
    ed.wait()

    def unpack(i, carry):
        for k in range(CH // 16):
            p = src_v[i, pl.ds(16 * k, 16)]
            src_v[i, pl.ds(16 * k, 16)] = lax.bitwise_and(p, 0xFFFF)
            dst_v[i, pl.ds(16 * k, 16)] = lax.shift_right_logical(p, 16)
        return carry

    lax.fori_loop(0, PERW, unpack, 0, unroll=False)
    hd.wait()

    plsc.subcore_barrier()

    def fire_half(g, off):
        for b in range(HB):
            pltpu.async_copy(h_sh.at[src_v.at[g * HB + b]],
                             rows_a.at[off + b], gsem.at[off + b])

    def drain_half(g, off):
        sds = []
        for b in range(HB):
            pltpu.make_async_copy(h_sh.at[src_v.at[g * HB + b]],
                                  rows_a.at[off + b],
                                  gsem.at[off + b]).wait()
            sds.append(
                pltpu.async_copy(rows_a.at[off + b],
                                 acc_sh.at[dst_v.at[g * HB + b]],
                                 sem, add=True))
        return sds

    NG2 = PERW // KB
    fire_half(0, 0)

    def ring(j, carry):
        fire_half(2 * j + 1, HB)
        for sd in drain_half(2 * j, 0):
            sd.wait()

        @pl.when(j < NG2 - 1)
        def _():
            fire_half(2 * j + 2, 0)

        for sd in drain_half(2 * j + 1, HB):
            sd.wait()
        return carry

    lax.fori_loop(0, NG2, ring, 0, unroll=False)

    plsc.subcore_barrier()

    pltpu.sync_copy(acc_sh.at[pl.ds(s * RPT, RPT)],
                    out_hbm.at[c, pl.ds(s * RPT, RPT)])


@functools.partial(jax.jit, static_argnums=())
def _segsum(h, edges):
    mesh = plsc.VectorSubcoreMesh(
        core_axis_name="c", subcore_axis_name="s",
        num_cores=NC, num_subcores=NS)
    fn = pl.kernel(
        _segsum_body,
        out_type=jax.ShapeDtypeStruct((NC, NP, D), jnp.float32),
        mesh=mesh,
        scratch_types=[
            pltpu.VMEM_SHARED((NP, D), jnp.float32),
            pltpu.VMEM_SHARED((NP, D), jnp.float32),
            pltpu.VMEM((PERW, CH), jnp.int32),
            pltpu.VMEM((PERW, CH), jnp.int32),
            pltpu.VMEM((KB, CH, D), jnp.float32),
            pltpu.VMEM((CH, D), jnp.float32),
            pltpu.SemaphoreType.DMA((KB,)),
            pltpu.SemaphoreType.DMA((2,)),
            pltpu.SemaphoreType.DMA,
        ],
        compiler_params=pltpu.CompilerParams(use_tc_tiling_on_sc=False),
    )
    return fn(h, edges)


PK = 4
PR = N // PK
PRP = NP // PK


def _fold_norm_relu(u, dh, g_t, be_t):
    L = u.shape[1]
    us = u[:PR]
    csum = jnp.sum(us, axis=0, keepdims=True)
    ii = lax.broadcasted_iota(jnp.int32, (L, L), 0) % dh
    jj = lax.broadcasted_iota(jnp.int32, (L, L), 1) % dh
    fold = (ii == jj).astype(jnp.float32)
    mu = jnp.dot(csum, fold, preferred_element_type=jnp.float32) / N
    d = u - mu
    ds = d[:PR]
    c2 = jnp.sum(ds * ds, axis=0, keepdims=True)
    var = jnp.dot(c2, fold, preferred_element_type=jnp.float32) / N
    return jnp.maximum(d / jnp.sqrt(var + 1e-5) * g_t + be_t, 0.0)


def _proj_body(x_ref, w_ref, o_ref):
    o_ref[...] = jnp.dot(x_ref[...], w_ref[...],
                         preferred_element_type=jnp.float32)


def _proj(x_pad, w_bd):
    return pl.pallas_call(
        _proj_body,
        out_shape=jax.ShapeDtypeStruct((PRP, PK * D), jnp.float32),
    )(x_pad, w_bd)


def _mlp1_body(y_ref, agg_ref, eps_ref, ba_ref, g_ref, be_ref, wb_ref,
               bb_ref, o_ref):
    u = ((1.0 + eps_ref[0, 0]) * y_ref[...] + agg_ref[0] + agg_ref[1]
         + ba_ref[...])
    h = _fold_norm_relu(u, D, g_ref[...], be_ref[...])
    o_ref[...] = jnp.dot(h, wb_ref[...],
                         preferred_element_type=jnp.float32) + bb_ref[...]


def _mlp1(y, agg, eps, ba_t, g_t, be_t, wb_bd, bb_t):
    return pl.pallas_call(
        _mlp1_body,
        out_shape=jax.ShapeDtypeStruct((PRP, PK * D), jnp.float32),
    )(y, agg, eps.reshape(1, 1), ba_t, g_t, be_t, wb_bd, bb_t)


def _mlp_body(h_ref, agg_ref, eps_ref, wa_ref, ba_ref, g_ref, be_ref,
              wb_ref, bb_ref, o_ref):
    t = (1.0 + eps_ref[0, 0]) * h_ref[...] + agg_ref[0] + agg_ref[1]
    y = jnp.dot(t, wa_ref[...],
                preferred_element_type=jnp.float32) + ba_ref[...]
    h = _fold_norm_relu(y, 64, g_ref[...], be_ref[...])
    o_ref[...] = jnp.dot(h, wb_ref[...],
                         preferred_element_type=jnp.float32) + bb_ref[...]


def _mlp(h, agg, eps, wa_bd, ba_t, g_t, be_t, wb_bd, bb_t):
    return pl.pallas_call(
        _mlp_body,
        out_shape=jax.ShapeDtypeStruct((PRP, PK * D), jnp.float32),
    )(h, agg, eps.reshape(1, 1), wa_bd, ba_t, g_t, be_t, wb_bd, bb_t)


def _final_body(h_ref, agg_ref, eps_ref, wa_ref, ba_ref, g_ref, be_ref,
                wb_ref, bb_ref, batch_ref, wl_ref, bl_ref, o_ref):
    t = (1.0 + eps_ref[0, 0]) * h_ref[...] + agg_ref[0] + agg_ref[1]
    y = jnp.dot(t, wa_ref[...],
                preferred_element_type=jnp.float32) + ba_ref[...]
    h = _fold_norm_relu(y, 64, g_ref[...], be_ref[...])
    h4 = jnp.dot(h, wb_ref[...],
                 preferred_element_type=jnp.float32) + bb_ref[...]
    gids = lax.broadcasted_iota(jnp.int32, (PR, G), 1)
    sums = jnp.zeros((G, 16), jnp.float32)
    counts = jnp.zeros((G, 1), jnp.float32)
    for k in range(PK):
        oh = (batch_ref[:, k:k + 1] == gids).astype(jnp.float32)
        sums = sums + lax.dot_general(
            oh, h4[:PR, 16 * k:16 * k + 16], (((0,), (0,)), ((), ())),
            preferred_element_type=jnp.float32)
        counts = counts + jnp.sum(oh, axis=0)[:, None]
    pooled = sums / jnp.maximum(counts, 1.0)
    logit = jnp.dot(pooled, wl_ref[...],
                    preferred_element_type=jnp.float32) + bl_ref[...]
    o_ref[...] = jax.nn.sigmoid(logit)


def _final(h, agg, eps, wa_bd, ba_t, g_t, be_t, wb_bd, bb_t, batch_p,
           Wl, bl):
    return pl.pallas_call(
        _final_body,
        out_shape=jax.ShapeDtypeStruct((G, 1), jnp.float32),
    )(h, agg, eps.reshape(1, 1), wa_bd, ba_t, g_t, be_t, wb_bd, bb_t,
      batch_p, Wl, bl.reshape(1, 1))


def _pack_body(e_ref, o_ref):
    o_ref[2496:] = jnp.full((NCHT - 2496, CH), N << 16, jnp.int32)
    o_ref[:2500] = jnp.bitwise_or(e_ref[0], jnp.left_shift(e_ref[1], 16))


def _pack(edge_index):
    return pl.pallas_call(
        _pack_body,
        out_shape=jax.ShapeDtypeStruct((NCHT, CH), jnp.int32),
    )(edge_index.reshape(2, E // CH, CH))


def _bd(W):
    return jnp.kron(jnp.eye(PK, dtype=jnp.float32), W)


def _t4(v):
    return jnp.tile(v, PK)[None, :]


def kernel(x, edge_index, batch, eps1, Wa1, ba1, g1, be1, Wb1, bb1,
           eps2, Wa2, ba2, g2, be2, Wb2, bb2,
           eps3, Wa3, ba3, g3, be3, Wb3, bb3,
           eps4, Wa4, ba4, g4, be4, Wb4, bb4, Wl, bl):
    edges = _pack(edge_index)
    x_pad = jnp.concatenate(
        [x, jnp.zeros((NP - N, x.shape[1]), jnp.float32)]).reshape(PRP, -1)
    batch_p = batch.reshape(PR, PK)

    y1 = _proj(x_pad, _bd(Wa1))
    a1 = _segsum(y1.reshape(NP, D), edges)
    h1 = _mlp1(y1, a1.reshape(NC, PRP, PK * D), eps1, _t4(ba1), _t4(g1),
               _t4(be1), _bd(Wb1), _t4(bb1))

    a2 = _segsum(h1.reshape(NP, D), edges)
    h2 = _mlp(h1, a2.reshape(NC, PRP, PK * D), eps2, _bd(Wa2), _t4(ba2),
              _t4(g2), _t4(be2), _bd(Wb2), _t4(bb2))

    a3 = _segsum(h2.reshape(NP, D), edges)
    h3 = _mlp(h2, a3.reshape(NC, PRP, PK * D), eps3, _bd(Wa3), _t4(ba3),
              _t4(g3), _t4(be3), _bd(Wb3), _t4(bb3))

    a4 = _segsum(h3.reshape(NP, D), edges)
    return _final(h3, a4.reshape(NC, PRP, PK * D), eps4, _bd(Wa4),
                  _t4(ba4), _t4(g4), _t4(be4), _bd(Wb4), _t4(bb4),
                  batch_p, Wl, bl)
